# baseline (speedup 1.0000x reference)
import dataclasses
import functools

import jax
import jax.numpy as jnp
from jax import lax
from jax.experimental import pallas as pl
from jax.experimental.pallas import tpu as pltpu
from jax.experimental.pallas import tpu_sc as plsc

N = 10000
D = 128
E = 320000
NC = 2
NS = 16
NW = NC * NS
CHUNK = 64
PCH = 128
CPT = 160
HCPT = 32
E_PAD = NW * CPT * CHUNK
ROWS = 10240
RPT = ROWS // NS
TCB = 1000


def _sc_mesh():
    return plsc.VectorSubcoreMesh(core_axis_name="c", subcore_axis_name="s")


def _sc_params():
    cp = pltpu.CompilerParams()
    if "needs_layout_passes" in pltpu.CompilerParams.__dataclass_fields__:
        cp = dataclasses.replace(cp, needs_layout_passes=False)
    return cp


def _deg_partials(col_flat):
    @functools.partial(
        pl.kernel,
        out_type=jax.ShapeDtypeStruct((NW, ROWS), jnp.float32),
        mesh=_sc_mesh(),
        compiler_params=_sc_params(),
        scratch_types=[
            pltpu.VMEM((CPT * CHUNK,), jnp.int32),
            pltpu.VMEM((ROWS,), jnp.float32),
            pltpu.SemaphoreType.DMA,
        ],
    )
    def k(col_hbm, out_hbm, colv, hist, sem):
        wid = lax.axis_index("s") * NC + lax.axis_index("c")
        pltpu.async_copy(col_hbm.at[wid], colv, sem).wait()
        zero = jnp.zeros((16,), jnp.float32)
        ones = jnp.ones((16,), jnp.float32)

        @pl.loop(0, ROWS // 16)
        def _(i):
            hist[pl.ds(i * 16, 16)] = zero

        @pl.loop(0, (CPT * CHUNK) // 16)
        def _(i):
            idx = colv[pl.ds(i * 16, 16)]
            plsc.addupdate_scatter(hist, [idx], ones)

        pltpu.async_copy(hist, out_hbm.at[wid], sem).wait()

    return k(col_flat)


def _matmul(x, W):
    def body(xr, wr, outr):
        outr[...] = lax.dot_general(
            xr[...], wr[...], (((1,), (0,)), ((), ())),
            precision=lax.Precision.HIGHEST,
            preferred_element_type=jnp.float32,
        )

    return pl.pallas_call(
        body,
        grid=(N // TCB,),
        in_specs=[
            pl.BlockSpec((TCB, D), lambda i: (i, 0)),
            pl.BlockSpec((D, D), lambda i: (0, 0)),
        ],
        out_specs=pl.BlockSpec((TCB, D), lambda i: (i, 0)),
        out_shape=jax.ShapeDtypeStruct((N, D), jnp.float32),
    )(x, W)


def _dis(deg_partials):
    def body(dr, outr):
        deg = jnp.sum(dr[...], axis=0) + 1.0
        outr[...] = lax.rsqrt(deg)[:, None]

    return pl.pallas_call(
        body,
        out_shape=jax.ShapeDtypeStruct((ROWS, 1), jnp.float32),
    )(deg_partials)


def _scale(h, dis):
    def body(hr, dr, outr):
        outr[...] = jnp.broadcast_to((hr[...] * dr[...])[None], (NC, TCB, D))

    return pl.pallas_call(
        body,
        grid=(N // TCB,),
        in_specs=[
            pl.BlockSpec((TCB, D), lambda i: (i, 0)),
            pl.BlockSpec((TCB, 1), lambda i: (i, 0)),
        ],
        out_specs=pl.BlockSpec((NC, TCB, D), lambda i: (0, i, 0)),
        out_shape=jax.ShapeDtypeStruct((NC, N, D), jnp.float32),
    )(h, dis)


def _edge_aggregate(hprime, row_idx, col_idx):
    @functools.partial(
        pl.kernel,
        out_type=jax.ShapeDtypeStruct((NC, ROWS, D), jnp.float32),
        mesh=_sc_mesh(),
        compiler_params=_sc_params(),
        scratch_types=[
            pltpu.VMEM((HCPT, CHUNK), jnp.int32),
            pltpu.VMEM((HCPT, CHUNK), jnp.int32),
            pltpu.VMEM((PCH, D), jnp.float32),
            pltpu.VMEM((PCH, D), jnp.float32),
            pltpu.VMEM_SHARED((ROWS, D), jnp.float32),
            pltpu.SemaphoreType.DMA,
            pltpu.SemaphoreType.DMA,
            pltpu.SemaphoreType.DMA,
            pltpu.SemaphoreType.DMA,
            pltpu.SemaphoreType.DMA,
            pltpu.SemaphoreType.DMA,
            pltpu.SemaphoreType.DMA,
            pltpu.SemaphoreType.DMA,
            pltpu.SemaphoreType.DMA,
        ],
    )
    def k(h_hbm, row_hbm, col_hbm, out_hbm, rowv, colv,
          gb0, gb1, acc, gs0, gs1, gs2, gs3, ss0, ss1, ss2, ss3, sem):
        cid = lax.axis_index("c")
        sid = lax.axis_index("s")
        wid = sid * NC + cid

        def buf(b):
            phys = gb0 if b < 2 else gb1
            return phys.at[pl.ds((b % 2) * CHUNK, CHUNK)]

        gs = (gs0, gs1, gs2, gs3)
        ss = (ss0, ss1, ss2, ss3)

        zero = jnp.zeros((16,), jnp.float32)

        @pl.loop(0, PCH)
        def _(r):
            @pl.loop(0, D // 16)
            def _(l):
                gb0[r, pl.ds(l * 16, 16)] = zero

        @pl.loop(0, RPT // PCH)
        def _(t):
            pltpu.sync_copy(
                gb0,
                acc.at[pl.ds(sid * RPT + t * PCH, PCH)],
            )

        def start_g(b, j):
            pltpu.async_copy(h_hbm.at[cid].at[rowv.at[j]], buf(b), gs[b])

        def wait_g(b):
            pltpu.make_async_copy(
                h_hbm.at[cid].at[rowv.at[0]], buf(b), gs[b]
            ).wait()

        def start_s(b, j):
            pltpu.async_copy(buf(b), acc.at[colv.at[j]], ss[b], add=True)

        def wait_s(b):
            pltpu.make_async_copy(buf(b), acc.at[colv.at[0]], ss[b]).wait()

        plsc.subcore_barrier()

        def run_block(w):
            for half in range(CPT // HCPT):
                pltpu.async_copy(
                    row_hbm.at[w].at[pl.ds(half * HCPT, HCPT)], rowv, sem
                ).wait()
                pltpu.async_copy(
                    col_hbm.at[w].at[pl.ds(half * HCPT, HCPT)], colv, sem
                ).wait()
                start_g(0, 0)
                start_g(1, 1)

                @pl.loop(0, HCPT // 4)
                def _(t):
                    j = 4 * t
                    wait_g(0)
                    wait_g(1)

                    @pl.when(t > 0)
                    def _():
                        wait_s(2)
                        wait_s(3)

                    start_g(2, j + 2)
                    start_g(3, j + 3)
                    start_s(0, j)
                    start_s(1, j + 1)
                    wait_g(2)
                    wait_g(3)
                    wait_s(0)
                    wait_s(1)

                    @pl.when(t < HCPT // 4 - 1)
                    def _():
                        start_g(0, j + 4)
                        start_g(1, j + 5)

                    start_s(2, j + 2)
                    start_s(3, j + 3)

                wait_s(2)
                wait_s(3)

        run_block(wid)
        plsc.subcore_barrier()

        pltpu.async_copy(
            acc.at[pl.ds(sid * RPT, RPT)],
            out_hbm.at[cid].at[pl.ds(sid * RPT, RPT)],
            sem,
        ).wait()

    return k(hprime, row_idx, col_idx)


def _finalize(acc2, hprime, dis, b2, alpha2):
    def body(ar, hr, dr, br, alr, outr):
        s = ar[0] + ar[1] + hr[0]
        out = dr[...] * s + br[...]
        outr[...] = jnp.where(out > 0, out, alr[...] * out)

    return pl.pallas_call(
        body,
        grid=(N // TCB,),
        in_specs=[
            pl.BlockSpec((NC, TCB, D), lambda i: (0, i, 0)),
            pl.BlockSpec((1, TCB, D), lambda i: (0, i, 0)),
            pl.BlockSpec((TCB, 1), lambda i: (i, 0)),
            pl.BlockSpec((1, D), lambda i: (0, 0)),
            pl.BlockSpec((1, D), lambda i: (0, 0)),
        ],
        out_specs=pl.BlockSpec((TCB, D), lambda i: (i, 0)),
        out_shape=jax.ShapeDtypeStruct((N, D), jnp.float32),
    )(acc2, hprime, dis, b2, alpha2)


def kernel(x, edge_index, k_centers, W, b, alpha):
    pad = E_PAD - E
    row = jnp.concatenate([edge_index[0], jnp.zeros((pad,), jnp.int32)])
    col = jnp.concatenate([edge_index[1], jnp.full((pad,), N, jnp.int32)])
    row3 = row.reshape(NW, CPT, CHUNK)
    col3 = col.reshape(NW, CPT, CHUNK)
    col_flat = col.reshape(NW, CPT * CHUNK)

    degp = _deg_partials(col_flat)
    h = _matmul(x, W)
    dis = _dis(degp)
    hp = _scale(h, dis)
    acc2 = _edge_aggregate(hp, row3, col3)
    return _finalize(acc2, hp, dis, b.reshape(1, D), alpha.reshape(1, D))


# speedup vs baseline: 1.0819x; 1.0819x over previous
import dataclasses
import functools

import jax
import jax.numpy as jnp
from jax import lax
from jax.experimental import pallas as pl
from jax.experimental.pallas import tpu as pltpu
from jax.experimental.pallas import tpu_sc as plsc

N = 10000
D = 128
E = 320000
NC = 2
NS = 16
NW = NC * NS
CHUNK = 128
CPT = 80
HCPT = 16
E_PAD = NW * CPT * CHUNK
ROWS = 10240
RPT = ROWS // NS
TCB = 1000


def _sc_mesh():
    return plsc.VectorSubcoreMesh(core_axis_name="c", subcore_axis_name="s")


def _sc_params():
    cp = pltpu.CompilerParams()
    if "needs_layout_passes" in pltpu.CompilerParams.__dataclass_fields__:
        cp = dataclasses.replace(cp, needs_layout_passes=False)
    return cp


def _deg_partials(col_flat):
    @functools.partial(
        pl.kernel,
        out_type=jax.ShapeDtypeStruct((NW, ROWS), jnp.float32),
        mesh=_sc_mesh(),
        compiler_params=_sc_params(),
        scratch_types=[
            pltpu.VMEM((CPT * CHUNK,), jnp.int32),
            pltpu.VMEM((ROWS,), jnp.float32),
            pltpu.SemaphoreType.DMA,
        ],
    )
    def k(col_hbm, out_hbm, colv, hist, sem):
        wid = lax.axis_index("s") * NC + lax.axis_index("c")
        pltpu.async_copy(col_hbm.at[wid], colv, sem).wait()
        zero = jnp.zeros((16,), jnp.float32)
        ones = jnp.ones((16,), jnp.float32)

        @pl.loop(0, ROWS // 16)
        def _(i):
            hist[pl.ds(i * 16, 16)] = zero

        @pl.loop(0, (CPT * CHUNK) // 16)
        def _(i):
            idx = colv[pl.ds(i * 16, 16)]
            plsc.addupdate_scatter(hist, [idx], ones)

        pltpu.async_copy(hist, out_hbm.at[wid], sem).wait()

    return k(col_flat)


def _matmul(x, W):
    def body(xr, wr, outr):
        outr[...] = lax.dot_general(
            xr[...], wr[...], (((1,), (0,)), ((), ())),
            precision=lax.Precision.HIGHEST,
            preferred_element_type=jnp.float32,
        )

    return pl.pallas_call(
        body,
        grid=(N // TCB,),
        in_specs=[
            pl.BlockSpec((TCB, D), lambda i: (i, 0)),
            pl.BlockSpec((D, D), lambda i: (0, 0)),
        ],
        out_specs=pl.BlockSpec((TCB, D), lambda i: (i, 0)),
        out_shape=jax.ShapeDtypeStruct((N, D), jnp.float32),
    )(x, W)


def _dis(deg_partials):
    def body(dr, outr):
        deg = jnp.sum(dr[...], axis=0) + 1.0
        outr[...] = lax.rsqrt(deg)[:, None]

    return pl.pallas_call(
        body,
        out_shape=jax.ShapeDtypeStruct((ROWS, 1), jnp.float32),
    )(deg_partials)


def _scale(h, dis):
    def body(hr, dr, outr):
        outr[...] = jnp.broadcast_to((hr[...] * dr[...])[None], (NC, TCB, D))

    return pl.pallas_call(
        body,
        grid=(N // TCB,),
        in_specs=[
            pl.BlockSpec((TCB, D), lambda i: (i, 0)),
            pl.BlockSpec((TCB, 1), lambda i: (i, 0)),
        ],
        out_specs=pl.BlockSpec((NC, TCB, D), lambda i: (0, i, 0)),
        out_shape=jax.ShapeDtypeStruct((NC, N, D), jnp.float32),
    )(h, dis)


def _edge_aggregate(hprime, row_idx, col_idx):
    @functools.partial(
        pl.kernel,
        out_type=jax.ShapeDtypeStruct((NC, ROWS, D), jnp.float32),
        mesh=_sc_mesh(),
        compiler_params=_sc_params(),
        scratch_types=[
            pltpu.VMEM((HCPT, CHUNK), jnp.int32),
            pltpu.VMEM((HCPT, CHUNK), jnp.int32),
            pltpu.VMEM((CHUNK, D), jnp.float32),
            pltpu.VMEM((CHUNK, D), jnp.float32),
            pltpu.VMEM_SHARED((ROWS, D), jnp.float32),
            pltpu.SemaphoreType.DMA,
            pltpu.SemaphoreType.DMA,
            pltpu.SemaphoreType.DMA,
            pltpu.SemaphoreType.DMA,
            pltpu.SemaphoreType.DMA,
        ],
    )
    def k(h_hbm, row_hbm, col_hbm, out_hbm, rowv, colv,
          gb0, gb1, acc, gs0, gs1, ss0, ss1, sem):
        cid = lax.axis_index("c")
        sid = lax.axis_index("s")
        wid = sid * NC + cid

        def buf(b):
            return gb0 if b == 0 else gb1

        gs = (gs0, gs1)
        ss = (ss0, ss1)

        zero = jnp.zeros((16,), jnp.float32)

        @pl.loop(0, CHUNK)
        def _(r):
            @pl.loop(0, D // 16)
            def _(l):
                gb0[r, pl.ds(l * 16, 16)] = zero

        @pl.loop(0, RPT // CHUNK)
        def _(t):
            pltpu.sync_copy(
                gb0,
                acc.at[pl.ds(sid * RPT + t * CHUNK, CHUNK)],
            )

        def start_g(b, j):
            pltpu.async_copy(h_hbm.at[cid].at[rowv.at[j]], buf(b), gs[b])

        def wait_g(b):
            pltpu.make_async_copy(
                h_hbm.at[cid].at[rowv.at[0]], buf(b), gs[b]
            ).wait()

        def start_s(b, j):
            pltpu.async_copy(buf(b), acc.at[colv.at[j]], ss[b], add=True)

        def wait_s(b):
            pltpu.make_async_copy(buf(b), acc.at[colv.at[0]], ss[b]).wait()

        plsc.subcore_barrier()

        def run_block(w):
            for half in range(CPT // HCPT):
                pltpu.async_copy(
                    row_hbm.at[w].at[pl.ds(half * HCPT, HCPT)], rowv, sem
                ).wait()
                pltpu.async_copy(
                    col_hbm.at[w].at[pl.ds(half * HCPT, HCPT)], colv, sem
                ).wait()
                start_g(0, 0)

                @pl.loop(0, HCPT // 2)
                def _(t):
                    j = 2 * t
                    wait_g(0)

                    @pl.when(t > 0)
                    def _():
                        wait_s(1)

                    start_g(1, j + 1)
                    start_s(0, j)
                    wait_g(1)
                    wait_s(0)

                    @pl.when(t < HCPT // 2 - 1)
                    def _():
                        start_g(0, j + 2)

                    start_s(1, j + 1)

                wait_s(1)

        run_block(wid)
        plsc.subcore_barrier()

        pltpu.async_copy(
            acc.at[pl.ds(sid * RPT, RPT)],
            out_hbm.at[cid].at[pl.ds(sid * RPT, RPT)],
            sem,
        ).wait()

    return k(hprime, row_idx, col_idx)


def _finalize(acc2, hprime, dis, b2, alpha2):
    def body(ar, hr, dr, br, alr, outr):
        s = ar[0] + ar[1] + hr[0]
        out = dr[...] * s + br[...]
        outr[...] = jnp.where(out > 0, out, alr[...] * out)

    return pl.pallas_call(
        body,
        grid=(N // TCB,),
        in_specs=[
            pl.BlockSpec((NC, TCB, D), lambda i: (0, i, 0)),
            pl.BlockSpec((1, TCB, D), lambda i: (0, i, 0)),
            pl.BlockSpec((TCB, 1), lambda i: (i, 0)),
            pl.BlockSpec((1, D), lambda i: (0, 0)),
            pl.BlockSpec((1, D), lambda i: (0, 0)),
        ],
        out_specs=pl.BlockSpec((TCB, D), lambda i: (i, 0)),
        out_shape=jax.ShapeDtypeStruct((N, D), jnp.float32),
    )(acc2, hprime, dis, b2, alpha2)


def kernel(x, edge_index, k_centers, W, b, alpha):
    pad = E_PAD - E
    row = jnp.concatenate([edge_index[0], jnp.zeros((pad,), jnp.int32)])
    col = jnp.concatenate([edge_index[1], jnp.full((pad,), N, jnp.int32)])
    row3 = row.reshape(NW, CPT, CHUNK)
    col3 = col.reshape(NW, CPT, CHUNK)
    col_flat = col.reshape(NW, CPT * CHUNK)

    degp = _deg_partials(col_flat)
    h = _matmul(x, W)
    dis = _dis(degp)
    hp = _scale(h, dis)
    acc2 = _edge_aggregate(hp, row3, col3)
    return _finalize(acc2, hp, dis, b.reshape(1, D), alpha.reshape(1, D))


# speedup vs baseline: 1.0861x; 1.0039x over previous
import dataclasses
import functools

import jax
import jax.numpy as jnp
from jax import lax
from jax.experimental import pallas as pl
from jax.experimental.pallas import tpu as pltpu
from jax.experimental.pallas import tpu_sc as plsc

N = 10000
D = 128
E = 320000
NC = 2
NS = 16
NW = NC * NS
CHUNK = 128
CPT = 80
HCPT = 16
E_PAD = NW * CPT * CHUNK
ROWS = 10240
RPT = ROWS // NS
TCB = 1000


def _sc_mesh():
    return plsc.VectorSubcoreMesh(core_axis_name="c", subcore_axis_name="s")


def _sc_params():
    cp = pltpu.CompilerParams()
    if "needs_layout_passes" in pltpu.CompilerParams.__dataclass_fields__:
        cp = dataclasses.replace(cp, needs_layout_passes=False)
    return cp


def _deg_partials(col_flat):
    @functools.partial(
        pl.kernel,
        out_type=jax.ShapeDtypeStruct((NW, ROWS), jnp.float32),
        mesh=_sc_mesh(),
        compiler_params=_sc_params(),
        scratch_types=[
            pltpu.VMEM((CPT * CHUNK,), jnp.int32),
            pltpu.VMEM((ROWS,), jnp.float32),
            pltpu.SemaphoreType.DMA,
        ],
    )
    def k(col_hbm, out_hbm, colv, hist, sem):
        wid = lax.axis_index("s") * NC + lax.axis_index("c")
        pltpu.async_copy(col_hbm.at[wid], colv, sem).wait()
        zero = jnp.zeros((16,), jnp.float32)
        ones = jnp.ones((16,), jnp.float32)

        @pl.loop(0, ROWS // 16)
        def _(i):
            hist[pl.ds(i * 16, 16)] = zero

        @pl.loop(0, (CPT * CHUNK) // 16)
        def _(i):
            idx = colv[pl.ds(i * 16, 16)]
            plsc.addupdate_scatter(hist, [idx], ones)

        pltpu.async_copy(hist, out_hbm.at[wid], sem).wait()

    return k(col_flat)


def _matmul(x, W):
    def body(xr, wr, outr):
        outr[...] = lax.dot_general(
            xr[...], wr[...], (((1,), (0,)), ((), ())),
            precision=lax.Precision.HIGHEST,
            preferred_element_type=jnp.float32,
        )

    return pl.pallas_call(
        body,
        grid=(N // TCB,),
        in_specs=[
            pl.BlockSpec((TCB, D), lambda i: (i, 0)),
            pl.BlockSpec((D, D), lambda i: (0, 0)),
        ],
        out_specs=pl.BlockSpec((TCB, D), lambda i: (i, 0)),
        out_shape=jax.ShapeDtypeStruct((N, D), jnp.float32),
    )(x, W)


def _dis(deg_partials):
    def body(dr, outr):
        deg = jnp.sum(dr[...], axis=0) + 1.0
        outr[...] = lax.rsqrt(deg)[:, None]

    return pl.pallas_call(
        body,
        out_shape=jax.ShapeDtypeStruct((ROWS, 1), jnp.float32),
    )(deg_partials)


def _scale(h, dis):
    def body(hr, dr, outr):
        outr[...] = jnp.broadcast_to((hr[...] * dr[...])[None], (NC, TCB, D))

    return pl.pallas_call(
        body,
        grid=(N // TCB,),
        in_specs=[
            pl.BlockSpec((TCB, D), lambda i: (i, 0)),
            pl.BlockSpec((TCB, 1), lambda i: (i, 0)),
        ],
        out_specs=pl.BlockSpec((NC, TCB, D), lambda i: (0, i, 0)),
        out_shape=jax.ShapeDtypeStruct((NC, N, D), jnp.float32),
    )(h, dis)


def _edge_aggregate(hprime, row_idx, col_idx):
    @functools.partial(
        pl.kernel,
        out_type=jax.ShapeDtypeStruct((NC, ROWS, D), jnp.float32),
        mesh=_sc_mesh(),
        compiler_params=_sc_params(),
        scratch_types=[
            pltpu.VMEM((HCPT, CHUNK), jnp.int32),
            pltpu.VMEM((HCPT, CHUNK), jnp.int32),
            pltpu.VMEM((CHUNK, D), jnp.float32),
            pltpu.VMEM((CHUNK, D), jnp.float32),
            pltpu.VMEM_SHARED((ROWS, D), jnp.float32),
            pltpu.SemaphoreType.DMA,
            pltpu.SemaphoreType.DMA,
            pltpu.SemaphoreType.DMA,
            pltpu.SemaphoreType.DMA,
            pltpu.SemaphoreType.DMA,
        ],
    )
    def k(h_hbm, row_hbm, col_hbm, out_hbm, rowv, colv,
          gb0, gb1, acc, gs0, gs1, ss0, ss1, sem):
        cid = lax.axis_index("c")
        sid = lax.axis_index("s")
        wid = sid * NC + cid

        def buf(b):
            return gb0 if b == 0 else gb1

        gs = (gs0, gs1)
        ss = (ss0, ss1)

        zero = jnp.zeros((16,), jnp.float32)

        @pl.loop(0, CHUNK)
        def _(r):
            @pl.loop(0, D // 16)
            def _(l):
                gb0[r, pl.ds(l * 16, 16)] = zero

        for t in range(RPT // CHUNK):
            pltpu.async_copy(
                gb0, acc.at[pl.ds(sid * RPT + t * CHUNK, CHUNK)], sem
            )
        for t in range(RPT // CHUNK):
            pltpu.make_async_copy(
                gb0, acc.at[pl.ds(sid * RPT + t * CHUNK, CHUNK)], sem
            ).wait()

        def start_g(b, j):
            pltpu.async_copy(h_hbm.at[cid].at[rowv.at[j]], buf(b), gs[b])

        def wait_g(b):
            pltpu.make_async_copy(
                h_hbm.at[cid].at[rowv.at[0]], buf(b), gs[b]
            ).wait()

        def start_s(b, j):
            pltpu.async_copy(buf(b), acc.at[colv.at[j]], ss[b], add=True)

        def wait_s(b):
            pltpu.make_async_copy(buf(b), acc.at[colv.at[0]], ss[b]).wait()

        plsc.subcore_barrier()

        def run_block(w):
            for half in range(CPT // HCPT):
                pltpu.async_copy(
                    row_hbm.at[w].at[pl.ds(half * HCPT, HCPT)], rowv, sem
                )
                pltpu.async_copy(
                    col_hbm.at[w].at[pl.ds(half * HCPT, HCPT)], colv, sem
                )
                pltpu.make_async_copy(
                    row_hbm.at[w].at[pl.ds(half * HCPT, HCPT)], rowv, sem
                ).wait()
                pltpu.make_async_copy(
                    col_hbm.at[w].at[pl.ds(half * HCPT, HCPT)], colv, sem
                ).wait()
                start_g(0, 0)

                @pl.loop(0, HCPT // 2)
                def _(t):
                    j = 2 * t
                    wait_g(0)

                    @pl.when(t > 0)
                    def _():
                        wait_s(1)

                    start_g(1, j + 1)
                    start_s(0, j)
                    wait_g(1)
                    wait_s(0)

                    @pl.when(t < HCPT // 2 - 1)
                    def _():
                        start_g(0, j + 2)

                    start_s(1, j + 1)

                wait_s(1)

        run_block(wid)
        plsc.subcore_barrier()

        pltpu.async_copy(
            acc.at[pl.ds(sid * RPT, RPT)],
            out_hbm.at[cid].at[pl.ds(sid * RPT, RPT)],
            sem,
        ).wait()

    return k(hprime, row_idx, col_idx)


def _finalize(acc2, hprime, dis, b2, alpha2):
    def body(ar, hr, dr, br, alr, outr):
        s = ar[0] + ar[1] + hr[0]
        out = dr[...] * s + br[...]
        outr[...] = jnp.where(out > 0, out, alr[...] * out)

    return pl.pallas_call(
        body,
        grid=(N // TCB,),
        in_specs=[
            pl.BlockSpec((NC, TCB, D), lambda i: (0, i, 0)),
            pl.BlockSpec((1, TCB, D), lambda i: (0, i, 0)),
            pl.BlockSpec((TCB, 1), lambda i: (i, 0)),
            pl.BlockSpec((1, D), lambda i: (0, 0)),
            pl.BlockSpec((1, D), lambda i: (0, 0)),
        ],
        out_specs=pl.BlockSpec((TCB, D), lambda i: (i, 0)),
        out_shape=jax.ShapeDtypeStruct((N, D), jnp.float32),
    )(acc2, hprime, dis, b2, alpha2)


def kernel(x, edge_index, k_centers, W, b, alpha):
    pad = E_PAD - E
    row = jnp.concatenate([edge_index[0], jnp.zeros((pad,), jnp.int32)])
    col = jnp.concatenate([edge_index[1], jnp.full((pad,), N, jnp.int32)])
    row3 = row.reshape(NW, CPT, CHUNK)
    col3 = col.reshape(NW, CPT, CHUNK)
    col_flat = col.reshape(NW, CPT * CHUNK)

    degp = _deg_partials(col_flat)
    h = _matmul(x, W)
    dis = _dis(degp)
    hp = _scale(h, dis)
    acc2 = _edge_aggregate(hp, row3, col3)
    return _finalize(acc2, hp, dis, b.reshape(1, D), alpha.reshape(1, D))
